# Initial kernel scaffold; baseline (speedup 1.0000x reference)
import jax
import jax.numpy as jnp
from jax.experimental import pallas as pl


def kernel(pos_seq, particle_type, batch_idx, params):
    raise NotImplementedError("write your pallas kernel here")


# speedup vs baseline: 1.0063x; 1.0063x over previous
import functools

import jax
import jax.numpy as jnp
from jax.experimental import pallas as pl

_BLK = 1024
_H = 128


def _pad_rows(a, m):
    n = a.shape[0]
    p = (-n) % m
    if p:
        a = jnp.pad(a, ((0, p),) + ((0, 0),) * (a.ndim - 1))
    return a


def _pad_cols(a, m):
    c = a.shape[1]
    p = (-c) % m
    if p:
        a = jnp.pad(a, ((0, 0), (0, p)))
    return a


def _mlp_body(nlayers, x_ref, *refs):
    out_ref = refs[-1]
    h = x_ref[...]
    for i in range(nlayers):
        w = refs[2 * i][...]
        b = refs[2 * i + 1][...]
        h = jnp.dot(h, w, preferred_element_type=jnp.float32) + b
        if i < nlayers - 1:
            h = jnp.maximum(h, 0.0)
    out_ref[...] = h


def _mlp_pallas(x, layers, blk=_BLK):
    n = x.shape[0]
    assert n % blk == 0
    nlayers = len(layers)
    flat = []
    in_specs = [pl.BlockSpec((blk, x.shape[1]), lambda i: (i, 0))]
    for w, b in layers:
        flat += [w, b.reshape(1, -1)]
        in_specs += [
            pl.BlockSpec(w.shape, lambda i: (0, 0)),
            pl.BlockSpec((1, b.shape[0]), lambda i: (0, 0)),
        ]
    out_dim = layers[-1][0].shape[1]
    return pl.pallas_call(
        functools.partial(_mlp_body, nlayers),
        grid=(n // blk,),
        in_specs=in_specs,
        out_specs=pl.BlockSpec((blk, out_dim), lambda i: (i, 0)),
        out_shape=jax.ShapeDtypeStruct((n, out_dim), jnp.float32),
    )(x, *flat)


def _edge_body(xs_ref, xd_ref, ea_ref, wa, wb, wc, b1, w2, b2, w3, b3, w4,
               b4, msg_ref, nea_ref):
    h = (jnp.dot(xs_ref[...], wa[...], preferred_element_type=jnp.float32)
         + jnp.dot(xd_ref[...], wb[...], preferred_element_type=jnp.float32)
         + jnp.dot(ea_ref[...], wc[...], preferred_element_type=jnp.float32)
         + b1[...])
    h = jnp.maximum(h, 0.0)
    h = jnp.maximum(jnp.dot(h, w2[...], preferred_element_type=jnp.float32) + b2[...], 0.0)
    h = jnp.maximum(jnp.dot(h, w3[...], preferred_element_type=jnp.float32) + b3[...], 0.0)
    m = jnp.dot(h, w4[...], preferred_element_type=jnp.float32) + b4[...]
    msg_ref[...] = m
    nea_ref[...] = ea_ref[...] + m


def _edge_layer_pallas(xs, xd, ea, mlp, blk=_BLK):
    e = xs.shape[0]
    assert e % blk == 0
    (w1, b1), (w2, b2), (w3, b3), (w4, b4) = mlp
    wa, wb, wc = w1[:_H], w1[_H:2 * _H], w1[2 * _H:]
    ins = [xs, xd, ea, wa, wb, wc, b1.reshape(1, -1), w2, b2.reshape(1, -1),
           w3, b3.reshape(1, -1), w4, b4.reshape(1, -1)]
    in_specs = [pl.BlockSpec((blk, _H), lambda i: (i, 0))] * 3 + [
        pl.BlockSpec(a.shape, lambda i: (0, 0)) for a in ins[3:]
    ]
    return pl.pallas_call(
        _edge_body,
        grid=(e // blk,),
        in_specs=in_specs,
        out_specs=[pl.BlockSpec((blk, _H), lambda i: (i, 0))] * 2,
        out_shape=[jax.ShapeDtypeStruct((e, _H), jnp.float32)] * 2,
    )(*ins)


def _node_body(x_ref, ag_ref, wa, wb, b1, w2, b2, w3, b3, w4, b4, out_ref):
    h = (jnp.dot(x_ref[...], wa[...], preferred_element_type=jnp.float32)
         + jnp.dot(ag_ref[...], wb[...], preferred_element_type=jnp.float32)
         + b1[...])
    h = jnp.maximum(h, 0.0)
    h = jnp.maximum(jnp.dot(h, w2[...], preferred_element_type=jnp.float32) + b2[...], 0.0)
    h = jnp.maximum(jnp.dot(h, w3[...], preferred_element_type=jnp.float32) + b3[...], 0.0)
    m = jnp.dot(h, w4[...], preferred_element_type=jnp.float32) + b4[...]
    out_ref[...] = x_ref[...] + m


def _node_layer_pallas(x, aggr, mlp, blk=_BLK):
    n = x.shape[0]
    assert n % blk == 0
    (w1, b1), (w2, b2), (w3, b3), (w4, b4) = mlp
    wa, wb = w1[:_H], w1[_H:]
    ins = [x, aggr, wa, wb, b1.reshape(1, -1), w2, b2.reshape(1, -1),
           w3, b3.reshape(1, -1), w4, b4.reshape(1, -1)]
    in_specs = [pl.BlockSpec((blk, _H), lambda i: (i, 0))] * 2 + [
        pl.BlockSpec(a.shape, lambda i: (0, 0)) for a in ins[2:]
    ]
    return pl.pallas_call(
        _node_body,
        grid=(n // blk,),
        in_specs=in_specs,
        out_specs=pl.BlockSpec((blk, _H), lambda i: (i, 0)),
        out_shape=jax.ShapeDtypeStruct((n, _H), jnp.float32),
    )(*ins)


def _pad_mlp(layers, in_pad):
    out = [(w, b) for (w, b) in layers]
    w0, b0 = out[0]
    if w0.shape[0] < in_pad:
        w0 = jnp.pad(w0, ((0, in_pad - w0.shape[0]), (0, 0)))
    out[0] = (w0, b0)
    wl, bl = out[-1]
    if wl.shape[1] < _H:
        wl = jnp.pad(wl, ((0, 0), (0, _H - wl.shape[1])))
        bl = jnp.pad(bl, (0, _H - bl.shape[0]))
    out[-1] = (wl, bl)
    return out


def kernel(pos_seq, particle_type, batch_idx, params):
    n = pos_seq.shape[0]
    r = 0.03
    e_cap = 60 * n

    velocities = pos_seq[:, 1:, :] - pos_seq[:, :-1, :]
    node_features = velocities.reshape(n, -1)
    type_emb = params["type_emb"][particle_type]
    node_input = jnp.concatenate([node_features, type_emb], axis=1)
    node_input = _pad_cols(_pad_rows(node_input, _BLK), _H)

    x = _mlp_pallas(node_input, _pad_mlp(params["node_enc"], _H))

    curr_pos = pos_seq[:, -1]
    sq = jnp.sum(curr_pos * curr_pos, axis=1)
    d2 = sq[:, None] + sq[None, :] - 2.0 * (curr_pos @ curr_pos.T)
    adj = d2 < (r * r)
    adj = adj & (batch_idx[:, None] == batch_idx[None, :])
    diag = jnp.arange(n)
    adj = adj.at[diag, diag].set(False)
    src, dst = jnp.nonzero(adj, size=e_cap, fill_value=n)

    e_pad = ((e_cap + _BLK - 1) // _BLK) * _BLK
    src = jnp.pad(src, (0, e_pad - e_cap), constant_values=n)
    dst = jnp.pad(dst, (0, e_pad - e_cap), constant_values=n)

    cp = _pad_rows(curr_pos, _BLK)
    rel_pos = cp[src] - cp[dst]
    edge_dist = jnp.sqrt(jnp.sum(rel_pos * rel_pos, axis=-1, keepdims=True))
    edge_input = _pad_cols(jnp.concatenate([rel_pos, edge_dist], axis=1), _H)

    edge_attr = _mlp_pallas(edge_input, _pad_mlp(params["edge_enc"], _H))

    n_pad = x.shape[0]
    for layer in params["gn"]:
        xs = x[src]
        xd = x[dst]
        msg, edge_attr = _edge_layer_pallas(xs, xd, edge_attr, layer["edge_mlp"])
        aggr = jax.ops.segment_sum(msg, dst, num_segments=n_pad)
        x = _node_layer_pallas(x, aggr, layer["node_mlp"])

    pred = _mlp_pallas(x, _pad_mlp(params["decoder"], _H))
    return pred[:n, :2]


# speedup vs baseline: 1.1895x; 1.1820x over previous
import functools

import jax
import jax.numpy as jnp
from jax.experimental import pallas as pl

_BLK = 1024
_H = 128


def _pad_rows(a, m):
    n = a.shape[0]
    p = (-n) % m
    if p:
        a = jnp.pad(a, ((0, p),) + ((0, 0),) * (a.ndim - 1))
    return a


def _pad_cols(a, m):
    c = a.shape[1]
    p = (-c) % m
    if p:
        a = jnp.pad(a, ((0, 0), (0, p)))
    return a


def _mlp_body(nlayers, x_ref, *refs):
    out_ref = refs[-1]
    h = x_ref[...]
    for i in range(nlayers):
        w = refs[2 * i][...]
        b = refs[2 * i + 1][...]
        h = jnp.dot(h, w, preferred_element_type=jnp.float32) + b
        if i < nlayers - 1:
            h = jnp.maximum(h, 0.0)
    out_ref[...] = h


def _mlp_pallas(x, layers, blk=_BLK):
    n = x.shape[0]
    assert n % blk == 0
    nlayers = len(layers)
    flat = []
    in_specs = [pl.BlockSpec((blk, x.shape[1]), lambda i: (i, 0))]
    for w, b in layers:
        flat += [w, b.reshape(1, -1)]
        in_specs += [
            pl.BlockSpec(w.shape, lambda i: (0, 0)),
            pl.BlockSpec((1, b.shape[0]), lambda i: (0, 0)),
        ]
    out_dim = layers[-1][0].shape[1]
    return pl.pallas_call(
        functools.partial(_mlp_body, nlayers),
        grid=(n // blk,),
        in_specs=in_specs,
        out_specs=pl.BlockSpec((blk, out_dim), lambda i: (i, 0)),
        out_shape=jax.ShapeDtypeStruct((n, out_dim), jnp.float32),
    )(x, *flat)


def _edge_body(xs_ref, xd_ref, ea_ref, wa, wb, wc, b1, w2, b2, w3, b3, w4,
               b4, msg_ref, nea_ref):
    h = (jnp.dot(xs_ref[...], wa[...], preferred_element_type=jnp.float32)
         + jnp.dot(xd_ref[...], wb[...], preferred_element_type=jnp.float32)
         + jnp.dot(ea_ref[...], wc[...], preferred_element_type=jnp.float32)
         + b1[...])
    h = jnp.maximum(h, 0.0)
    h = jnp.maximum(jnp.dot(h, w2[...], preferred_element_type=jnp.float32) + b2[...], 0.0)
    h = jnp.maximum(jnp.dot(h, w3[...], preferred_element_type=jnp.float32) + b3[...], 0.0)
    m = jnp.dot(h, w4[...], preferred_element_type=jnp.float32) + b4[...]
    msg_ref[...] = m
    nea_ref[...] = ea_ref[...] + m


def _edge_layer_pallas(xs, xd, ea, mlp, blk=_BLK):
    e = xs.shape[0]
    assert e % blk == 0
    (w1, b1), (w2, b2), (w3, b3), (w4, b4) = mlp
    wa, wb, wc = w1[:_H], w1[_H:2 * _H], w1[2 * _H:]
    ins = [xs, xd, ea, wa, wb, wc, b1.reshape(1, -1), w2, b2.reshape(1, -1),
           w3, b3.reshape(1, -1), w4, b4.reshape(1, -1)]
    in_specs = [pl.BlockSpec((blk, _H), lambda i: (i, 0))] * 3 + [
        pl.BlockSpec(a.shape, lambda i: (0, 0)) for a in ins[3:]
    ]
    return pl.pallas_call(
        _edge_body,
        grid=(e // blk,),
        in_specs=in_specs,
        out_specs=[pl.BlockSpec((blk, _H), lambda i: (i, 0))] * 2,
        out_shape=[jax.ShapeDtypeStruct((e, _H), jnp.float32)] * 2,
    )(*ins)


def _node_body(x_ref, ag_ref, wa, wb, b1, w2, b2, w3, b3, w4, b4, out_ref):
    h = (jnp.dot(x_ref[...], wa[...], preferred_element_type=jnp.float32)
         + jnp.dot(ag_ref[...], wb[...], preferred_element_type=jnp.float32)
         + b1[...])
    h = jnp.maximum(h, 0.0)
    h = jnp.maximum(jnp.dot(h, w2[...], preferred_element_type=jnp.float32) + b2[...], 0.0)
    h = jnp.maximum(jnp.dot(h, w3[...], preferred_element_type=jnp.float32) + b3[...], 0.0)
    m = jnp.dot(h, w4[...], preferred_element_type=jnp.float32) + b4[...]
    out_ref[...] = x_ref[...] + m


def _node_layer_pallas(x, aggr, mlp, blk=_BLK):
    n = x.shape[0]
    assert n % blk == 0
    (w1, b1), (w2, b2), (w3, b3), (w4, b4) = mlp
    wa, wb = w1[:_H], w1[_H:]
    ins = [x, aggr, wa, wb, b1.reshape(1, -1), w2, b2.reshape(1, -1),
           w3, b3.reshape(1, -1), w4, b4.reshape(1, -1)]
    in_specs = [pl.BlockSpec((blk, _H), lambda i: (i, 0))] * 2 + [
        pl.BlockSpec(a.shape, lambda i: (0, 0)) for a in ins[2:]
    ]
    return pl.pallas_call(
        _node_body,
        grid=(n // blk,),
        in_specs=in_specs,
        out_specs=pl.BlockSpec((blk, _H), lambda i: (i, 0)),
        out_shape=jax.ShapeDtypeStruct((n, _H), jnp.float32),
    )(*ins)


_RADIUS = 1792
_BAND = 256 + 2 * _RADIUS


def _band_mask_body(n, rx_ref, ry_ref, cwx_ref, cwy_ref, m_ref):
    i = pl.program_id(0)
    rx = rx_ref[0]
    ry = ry_ref[0]
    cwx = cwx_ref[0]
    cwy = cwy_ref[0]
    rxb = rx.astype(jnp.bfloat16).astype(jnp.float32)
    ryb = ry.astype(jnp.bfloat16).astype(jnp.float32)
    cxb = cwx.astype(jnp.bfloat16).astype(jnp.float32)
    cyb = cwy.astype(jnp.bfloat16).astype(jnp.float32)
    dot = rxb * cxb + ryb * cyb
    sqr = rx * rx + ry * ry
    sqc = cwx * cwx + cwy * cwy
    d2 = (sqr + sqc) - 2.0 * dot
    src = (i * 256 - _RADIUS) + jax.lax.broadcasted_iota(jnp.int32, (256, _BAND), 1)
    dst = i * 256 + jax.lax.broadcasted_iota(jnp.int32, (256, _BAND), 0)
    ok = (d2 < (0.03 * 0.03)) & (src != dst) & (src >= 0) & (src < n) & (dst < n)
    m_ref[0] = ok.astype(jnp.int8)


def _band_mask_pallas(px, py, n):
    npad = px.shape[0]
    nb = npad // 256
    rx = px.reshape(nb, 256, 1)
    ry = py.reshape(nb, 256, 1)
    ext_x = jnp.pad(px, (_RADIUS, _BAND))
    ext_y = jnp.pad(py, (_RADIUS, _BAND))
    cwx = jnp.stack([jax.lax.slice(ext_x, (i * 256,), (i * 256 + _BAND,))
                     for i in range(nb)]).reshape(nb, 1, _BAND)
    cwy = jnp.stack([jax.lax.slice(ext_y, (i * 256,), (i * 256 + _BAND,))
                     for i in range(nb)]).reshape(nb, 1, _BAND)
    return pl.pallas_call(
        functools.partial(_band_mask_body, n),
        grid=(nb,),
        in_specs=[
            pl.BlockSpec((1, 256, 1), lambda i: (i, 0, 0)),
            pl.BlockSpec((1, 256, 1), lambda i: (i, 0, 0)),
            pl.BlockSpec((1, 1, _BAND), lambda i: (i, 0, 0)),
            pl.BlockSpec((1, 1, _BAND), lambda i: (i, 0, 0)),
        ],
        out_specs=pl.BlockSpec((1, 256, _BAND), lambda i: (i, 0, 0)),
        out_shape=jax.ShapeDtypeStruct((nb, 256, _BAND), jnp.int8),
    )(rx, ry, cwx, cwy)


def _pad_mlp(layers, in_pad):
    out = [(w, b) for (w, b) in layers]
    w0, b0 = out[0]
    if w0.shape[0] < in_pad:
        w0 = jnp.pad(w0, ((0, in_pad - w0.shape[0]), (0, 0)))
    out[0] = (w0, b0)
    wl, bl = out[-1]
    if wl.shape[1] < _H:
        wl = jnp.pad(wl, ((0, 0), (0, _H - wl.shape[1])))
        bl = jnp.pad(bl, (0, _H - bl.shape[0]))
    out[-1] = (wl, bl)
    return out


def kernel(pos_seq, particle_type, batch_idx, params):
    n = pos_seq.shape[0]
    r = 0.03
    e_cap = 60 * n

    curr_pos = pos_seq[:, -1]
    perm = jnp.argsort(curr_pos[:, 1])
    inv_perm = jnp.argsort(perm)
    n_pad = ((n + _BLK - 1) // _BLK) * _BLK

    cp_s = curr_pos[perm]
    cp_pad = jnp.full((n_pad, 2), 2.0, jnp.float32).at[:n].set(cp_s)

    velocities = pos_seq[:, 1:, :] - pos_seq[:, :-1, :]
    node_features = velocities.reshape(n, -1)
    type_emb = params["type_emb"][particle_type]
    node_input = jnp.concatenate([node_features, type_emb], axis=1)[perm]
    node_input = _pad_cols(_pad_rows(node_input, _BLK), _H)

    x = _mlp_pallas(node_input, _pad_mlp(params["node_enc"], _H))

    mask = _band_mask_pallas(cp_pad[:, 0], cp_pad[:, 1], n)
    fill_d = n_pad - 1
    fill_flat = fill_d * _BAND + (fill_d - (fill_d // 256) * 256 + _RADIUS)
    (flat,) = jnp.nonzero(mask.reshape(-1), size=e_cap, fill_value=fill_flat)
    flat = flat.astype(jnp.int32)
    dst = flat // _BAND
    src = (dst // 256) * 256 - _RADIUS + (flat - dst * _BAND)

    e_pad = ((e_cap + _BLK - 1) // _BLK) * _BLK
    src = jnp.pad(src, (0, e_pad - e_cap), constant_values=fill_d)
    dst = jnp.pad(dst, (0, e_pad - e_cap), constant_values=fill_d)

    cp = cp_pad
    rel_pos = cp[src] - cp[dst]
    edge_dist = jnp.sqrt(jnp.sum(rel_pos * rel_pos, axis=-1, keepdims=True))
    edge_input = _pad_cols(jnp.concatenate([rel_pos, edge_dist], axis=1), _H)

    edge_attr = _mlp_pallas(edge_input, _pad_mlp(params["edge_enc"], _H))

    for layer in params["gn"]:
        xs = x[src]
        xd = x[dst]
        msg, edge_attr = _edge_layer_pallas(xs, xd, edge_attr, layer["edge_mlp"])
        aggr = jax.ops.segment_sum(msg, dst, num_segments=n_pad)
        x = _node_layer_pallas(x, aggr, layer["node_mlp"])

    pred = _mlp_pallas(x, _pad_mlp(params["decoder"], _H))
    return pred[:n, :2][inv_perm]


# speedup vs baseline: 1.9270x; 1.6201x over previous
import functools

import jax
import jax.numpy as jnp
from jax.experimental import pallas as pl
from jax.experimental.pallas import tpu as pltpu

_BLK = 1024
_H = 128


def _pad_rows(a, m):
    n = a.shape[0]
    p = (-n) % m
    if p:
        a = jnp.pad(a, ((0, p),) + ((0, 0),) * (a.ndim - 1))
    return a


def _pad_cols(a, m):
    c = a.shape[1]
    p = (-c) % m
    if p:
        a = jnp.pad(a, ((0, 0), (0, p)))
    return a


def _mlp_body(nlayers, x_ref, *refs):
    out_ref = refs[-1]
    h = x_ref[...]
    for i in range(nlayers):
        w = refs[2 * i][...]
        b = refs[2 * i + 1][...]
        h = jnp.dot(h, w, preferred_element_type=jnp.float32) + b
        if i < nlayers - 1:
            h = jnp.maximum(h, 0.0)
    out_ref[...] = h


def _mlp_pallas(x, layers, blk=_BLK):
    n = x.shape[0]
    assert n % blk == 0
    nlayers = len(layers)
    flat = []
    in_specs = [pl.BlockSpec((blk, x.shape[1]), lambda i: (i, 0))]
    for w, b in layers:
        flat += [w, b.reshape(1, -1)]
        in_specs += [
            pl.BlockSpec(w.shape, lambda i: (0, 0)),
            pl.BlockSpec((1, b.shape[0]), lambda i: (0, 0)),
        ]
    out_dim = layers[-1][0].shape[1]
    return pl.pallas_call(
        functools.partial(_mlp_body, nlayers),
        grid=(n // blk,),
        in_specs=in_specs,
        out_specs=pl.BlockSpec((blk, out_dim), lambda i: (i, 0)),
        out_shape=jax.ShapeDtypeStruct((n, out_dim), jnp.float32),
    )(x, *flat)


def _edge_body(xs_ref, xd_ref, ea_ref, wa, wb, wc, b1, w2, b2, w3, b3, w4,
               b4, msg_ref, nea_ref):
    h = (jnp.dot(xs_ref[...], wa[...], preferred_element_type=jnp.float32)
         + jnp.dot(xd_ref[...], wb[...], preferred_element_type=jnp.float32)
         + jnp.dot(ea_ref[...], wc[...], preferred_element_type=jnp.float32)
         + b1[...])
    h = jnp.maximum(h, 0.0)
    h = jnp.maximum(jnp.dot(h, w2[...], preferred_element_type=jnp.float32) + b2[...], 0.0)
    h = jnp.maximum(jnp.dot(h, w3[...], preferred_element_type=jnp.float32) + b3[...], 0.0)
    m = jnp.dot(h, w4[...], preferred_element_type=jnp.float32) + b4[...]
    msg_ref[...] = m
    nea_ref[...] = ea_ref[...] + m


def _edge_layer_pallas(xs, xd, ea, mlp, blk=_BLK):
    e = xs.shape[0]
    assert e % blk == 0
    (w1, b1), (w2, b2), (w3, b3), (w4, b4) = mlp
    wa, wb, wc = w1[:_H], w1[_H:2 * _H], w1[2 * _H:]
    ins = [xs, xd, ea, wa, wb, wc, b1.reshape(1, -1), w2, b2.reshape(1, -1),
           w3, b3.reshape(1, -1), w4, b4.reshape(1, -1)]
    in_specs = [pl.BlockSpec((blk, _H), lambda i: (i, 0))] * 3 + [
        pl.BlockSpec(a.shape, lambda i: (0, 0)) for a in ins[3:]
    ]
    return pl.pallas_call(
        _edge_body,
        grid=(e // blk,),
        in_specs=in_specs,
        out_specs=[pl.BlockSpec((blk, _H), lambda i: (i, 0))] * 2,
        out_shape=[jax.ShapeDtypeStruct((e, _H), jnp.float32)] * 2,
    )(*ins)


def _split(v):
    hi = v.astype(jnp.bfloat16)
    lo = (v - hi.astype(jnp.float32)).astype(jnp.bfloat16)
    return hi, lo


def _gn_edge_body(n, swin_ref, sk_ref, sfirst_ref, sact_ref,
                  src_ref, dst_ref, ea_ref, xlo_ref, xhi_ref, xdw_ref,
                  wa, wb, wc, b1, w2, b2, w3, b3, w4, b4,
                  nea_ref, aggr_ref):
    b = pl.program_id(0)
    first = sfirst_ref[b]
    active = sact_ref[b]

    @pl.when(first == 1)
    def _():
        aggr_ref[...] = jnp.zeros((_BLK, _H), jnp.float32)

    @pl.when(active == 0)
    def _():
        nea_ref[...] = ea_ref[...]

    @pl.when(active == 1)
    def _():
        base_s = sk_ref[b] * _BLK
        base_d = swin_ref[b] * _BLK
        s = src_ref[...]
        d = dst_ref[...]
        srel = s - base_s
        drel = d - base_d
        ssrc = (srel == jax.lax.broadcasted_iota(jnp.int32, (_BLK, 2 * _BLK), 1)
                ).astype(jnp.bfloat16)
        sdst = (drel == jax.lax.broadcasted_iota(jnp.int32, (_BLK, _BLK), 1)
                ).astype(jnp.bfloat16)
        xw = jnp.concatenate([xlo_ref[...], xhi_ref[...]], axis=0)
        xwh, xwl = _split(xw)
        xdh, xdl = _split(xdw_ref[...])
        f32 = jnp.float32
        xs = (jnp.dot(ssrc, xwh, preferred_element_type=f32)
              + jnp.dot(ssrc, xwl, preferred_element_type=f32))
        xd = (jnp.dot(sdst, xdh, preferred_element_type=f32)
              + jnp.dot(sdst, xdl, preferred_element_type=f32))
        ea = ea_ref[...]
        h = (jnp.dot(xs, wa[...], preferred_element_type=f32)
             + jnp.dot(xd, wb[...], preferred_element_type=f32)
             + jnp.dot(ea, wc[...], preferred_element_type=f32) + b1[...])
        h = jnp.maximum(h, 0.0)
        h = jnp.maximum(jnp.dot(h, w2[...], preferred_element_type=f32) + b2[...], 0.0)
        h = jnp.maximum(jnp.dot(h, w3[...], preferred_element_type=f32) + b3[...], 0.0)
        m = jnp.dot(h, w4[...], preferred_element_type=f32) + b4[...]
        valid = (s < n) & (d < n)
        m = jnp.where(valid, m, 0.0)
        nea_ref[...] = ea + m
        mh, ml = _split(m)
        dn = (((0,), (0,)), ((), ()))
        contrib = (jax.lax.dot_general(sdst, mh, dn, preferred_element_type=f32)
                   + jax.lax.dot_general(sdst, ml, dn, preferred_element_type=f32))
        aggr_ref[...] += contrib


def _gn_edge_layer(src_w, dst_w, ea, x_ext, swin, sk, sfirst, sact, mlp, n,
                   n_pad):
    e2 = ea.shape[0]
    nb = e2 // _BLK
    (w1, b1), (w2, b2), (w3, b3), (w4, b4) = mlp
    wa, wb, wc = w1[:_H], w1[_H:2 * _H], w1[2 * _H:]
    weights = [wa, wb, wc, b1.reshape(1, -1), w2, b2.reshape(1, -1),
               w3, b3.reshape(1, -1), w4, b4.reshape(1, -1)]
    grid_spec = pltpu.PrefetchScalarGridSpec(
        num_scalar_prefetch=4,
        grid=(nb,),
        in_specs=[
            pl.BlockSpec((_BLK, 1), lambda b, swin, sk, sfi, sac: (b, 0)),
            pl.BlockSpec((_BLK, 1), lambda b, swin, sk, sfi, sac: (b, 0)),
            pl.BlockSpec((_BLK, _H), lambda b, swin, sk, sfi, sac: (b, 0)),
            pl.BlockSpec((_BLK, _H), lambda b, swin, sk, sfi, sac: (sk[b], 0)),
            pl.BlockSpec((_BLK, _H), lambda b, swin, sk, sfi, sac: (sk[b] + 1, 0)),
            pl.BlockSpec((_BLK, _H), lambda b, swin, sk, sfi, sac: (swin[b], 0)),
        ] + [pl.BlockSpec(w.shape, lambda b, swin, sk, sfi, sac: (0, 0))
             for w in weights],
        out_specs=[
            pl.BlockSpec((_BLK, _H), lambda b, swin, sk, sfi, sac: (b, 0)),
            pl.BlockSpec((_BLK, _H), lambda b, swin, sk, sfi, sac: (swin[b], 0)),
        ],
    )
    nea, aggr = pl.pallas_call(
        functools.partial(_gn_edge_body, n),
        grid_spec=grid_spec,
        out_shape=[
            jax.ShapeDtypeStruct((e2, _H), jnp.float32),
            jax.ShapeDtypeStruct((n_pad, _H), jnp.float32),
        ],
    )(swin, sk, sfirst, sact, src_w, dst_w, ea, x_ext, x_ext, x_ext, *weights)
    return nea, aggr


def _node_body(x_ref, ag_ref, wa, wb, b1, w2, b2, w3, b3, w4, b4, out_ref):
    h = (jnp.dot(x_ref[...], wa[...], preferred_element_type=jnp.float32)
         + jnp.dot(ag_ref[...], wb[...], preferred_element_type=jnp.float32)
         + b1[...])
    h = jnp.maximum(h, 0.0)
    h = jnp.maximum(jnp.dot(h, w2[...], preferred_element_type=jnp.float32) + b2[...], 0.0)
    h = jnp.maximum(jnp.dot(h, w3[...], preferred_element_type=jnp.float32) + b3[...], 0.0)
    m = jnp.dot(h, w4[...], preferred_element_type=jnp.float32) + b4[...]
    out_ref[...] = x_ref[...] + m


def _node_layer_pallas(x, aggr, mlp, blk=_BLK):
    n = x.shape[0]
    assert n % blk == 0
    (w1, b1), (w2, b2), (w3, b3), (w4, b4) = mlp
    wa, wb = w1[:_H], w1[_H:]
    ins = [x, aggr, wa, wb, b1.reshape(1, -1), w2, b2.reshape(1, -1),
           w3, b3.reshape(1, -1), w4, b4.reshape(1, -1)]
    in_specs = [pl.BlockSpec((blk, _H), lambda i: (i, 0))] * 2 + [
        pl.BlockSpec(a.shape, lambda i: (0, 0)) for a in ins[2:]
    ]
    return pl.pallas_call(
        _node_body,
        grid=(n // blk,),
        in_specs=in_specs,
        out_specs=pl.BlockSpec((blk, _H), lambda i: (i, 0)),
        out_shape=jax.ShapeDtypeStruct((n, _H), jnp.float32),
    )(*ins)


_RADIUS = 1792
_BAND = 256 + 2 * _RADIUS


def _band_mask_body(n, rx_ref, ry_ref, cwx_ref, cwy_ref, m_ref):
    i = pl.program_id(0)
    rx = rx_ref[0]
    ry = ry_ref[0]
    cwx = cwx_ref[0]
    cwy = cwy_ref[0]
    rxb = rx.astype(jnp.bfloat16).astype(jnp.float32)
    ryb = ry.astype(jnp.bfloat16).astype(jnp.float32)
    cxb = cwx.astype(jnp.bfloat16).astype(jnp.float32)
    cyb = cwy.astype(jnp.bfloat16).astype(jnp.float32)
    dot = rxb * cxb + ryb * cyb
    sqr = rx * rx + ry * ry
    sqc = cwx * cwx + cwy * cwy
    d2 = (sqr + sqc) - 2.0 * dot
    src = (i * 256 - _RADIUS) + jax.lax.broadcasted_iota(jnp.int32, (256, _BAND), 1)
    dst = i * 256 + jax.lax.broadcasted_iota(jnp.int32, (256, _BAND), 0)
    ok = (d2 < (0.03 * 0.03)) & (src != dst) & (src >= 0) & (src < n) & (dst < n)
    m_ref[0] = ok.astype(jnp.int8)


def _band_mask_pallas(px, py, n):
    npad = px.shape[0]
    nb = npad // 256
    rx = px.reshape(nb, 256, 1)
    ry = py.reshape(nb, 256, 1)
    ext_x = jnp.pad(px, (_RADIUS, _BAND))
    ext_y = jnp.pad(py, (_RADIUS, _BAND))
    cwx = jnp.stack([jax.lax.slice(ext_x, (i * 256,), (i * 256 + _BAND,))
                     for i in range(nb)]).reshape(nb, 1, _BAND)
    cwy = jnp.stack([jax.lax.slice(ext_y, (i * 256,), (i * 256 + _BAND,))
                     for i in range(nb)]).reshape(nb, 1, _BAND)
    return pl.pallas_call(
        functools.partial(_band_mask_body, n),
        grid=(nb,),
        in_specs=[
            pl.BlockSpec((1, 256, 1), lambda i: (i, 0, 0)),
            pl.BlockSpec((1, 256, 1), lambda i: (i, 0, 0)),
            pl.BlockSpec((1, 1, _BAND), lambda i: (i, 0, 0)),
            pl.BlockSpec((1, 1, _BAND), lambda i: (i, 0, 0)),
        ],
        out_specs=pl.BlockSpec((1, 256, _BAND), lambda i: (i, 0, 0)),
        out_shape=jax.ShapeDtypeStruct((nb, 256, _BAND), jnp.int8),
    )(rx, ry, cwx, cwy)


def _pad_mlp(layers, in_pad):
    out = [(w, b) for (w, b) in layers]
    w0, b0 = out[0]
    if w0.shape[0] < in_pad:
        w0 = jnp.pad(w0, ((0, in_pad - w0.shape[0]), (0, 0)))
    out[0] = (w0, b0)
    wl, bl = out[-1]
    if wl.shape[1] < _H:
        wl = jnp.pad(wl, ((0, 0), (0, _H - wl.shape[1])))
        bl = jnp.pad(bl, (0, _H - bl.shape[0]))
    out[-1] = (wl, bl)
    return out


def kernel(pos_seq, particle_type, batch_idx, params):
    n = pos_seq.shape[0]
    r = 0.03
    e_cap = 60 * n

    curr_pos = pos_seq[:, -1]
    perm = jnp.argsort(curr_pos[:, 1])
    inv_perm = jnp.argsort(perm)
    n_pad = ((n + _BLK - 1) // _BLK) * _BLK

    cp_s = curr_pos[perm]
    cp_pad = jnp.full((n_pad, 2), 2.0, jnp.float32).at[:n].set(cp_s)

    velocities = pos_seq[:, 1:, :] - pos_seq[:, :-1, :]
    node_features = velocities.reshape(n, -1)
    type_emb = params["type_emb"][particle_type]
    node_input = jnp.concatenate([node_features, type_emb], axis=1)[perm]
    node_input = _pad_cols(_pad_rows(node_input, _BLK), _H)

    x = _mlp_pallas(node_input, _pad_mlp(params["node_enc"], _H))

    mask = _band_mask_pallas(cp_pad[:, 0], cp_pad[:, 1], n)
    fill_d = n_pad - 1
    fill_flat = fill_d * _BAND + (fill_d - (fill_d // 256) * 256 + _RADIUS)
    (flat,) = jnp.nonzero(mask.reshape(-1), size=e_cap, fill_value=fill_flat)
    flat = flat.astype(jnp.int32)
    dst = flat // _BAND
    src = (dst // 256) * 256 - _RADIUS + (flat - dst * _BAND)

    e_pad = ((e_cap + _BLK - 1) // _BLK) * _BLK
    src = jnp.pad(src, (0, e_pad - e_cap), constant_values=fill_d)
    dst = jnp.pad(dst, (0, e_pad - e_cap), constant_values=fill_d)

    nwin = n_pad // _BLK
    bounds = jnp.searchsorted(dst, jnp.arange(0, n_pad + 1, _BLK, dtype=jnp.int32)).astype(jnp.int32)
    counts = bounds[1:] - bounds[:-1]
    pcounts = jnp.maximum((counts + _BLK - 1) // _BLK, 1) * _BLK
    offs = jnp.concatenate([jnp.zeros(1, jnp.int32), jnp.cumsum(pcounts).astype(jnp.int32)])
    e2 = e_pad + nwin * _BLK
    p = jnp.arange(e2, dtype=jnp.int32)
    w_of_p = jnp.clip(jnp.searchsorted(offs, p, side="right") - 1, 0, nwin - 1)
    local = p - offs[w_of_p]
    inb = local < counts[w_of_p]
    gidx = jnp.where(inb, bounds[w_of_p] + local, e_pad - 1)
    src_w = src[gidx].astype(jnp.int32)
    dst_w = dst[gidx].astype(jnp.int32)

    nb = e2 // _BLK
    swin = jnp.clip(
        jnp.searchsorted(offs, jnp.arange(0, e2, _BLK, dtype=jnp.int32), side="right") - 1,
        0, nwin - 1).astype(jnp.int32)
    min_src = src_w.reshape(nb, _BLK).min(axis=1)
    sk = jnp.clip(min_src // _BLK, 0, nwin - 1).astype(jnp.int32)
    sfirst = jnp.concatenate(
        [jnp.ones(1, jnp.int32), (swin[1:] != swin[:-1]).astype(jnp.int32)])
    sact = (min_src < n).astype(jnp.int32)

    rel_pos = cp_pad[src_w] - cp_pad[dst_w]
    edge_dist = jnp.sqrt(jnp.sum(rel_pos * rel_pos, axis=-1, keepdims=True))
    edge_input = jnp.pad(jnp.concatenate([rel_pos, edge_dist], axis=1),
                         ((0, 0), (0, 5)))

    edge_attr = _mlp_pallas(edge_input, _pad_mlp(params["edge_enc"], 8))

    src_c = src_w.reshape(e2, 1)
    dst_c = dst_w.reshape(e2, 1)
    for layer in params["gn"]:
        x_ext = jnp.pad(x, ((0, _BLK), (0, 0)))
        edge_attr, aggr = _gn_edge_layer(src_c, dst_c, edge_attr, x_ext,
                                         swin, sk, sfirst, sact,
                                         layer["edge_mlp"], n, n_pad)
        x = _node_layer_pallas(x, aggr, layer["node_mlp"])

    pred = _mlp_pallas(x, _pad_mlp(params["decoder"], _H))
    return pred[:n, :2][inv_perm]
